# Initial kernel scaffold; baseline (speedup 1.0000x reference)
import jax
import jax.numpy as jnp
from jax.experimental import pallas as pl


def kernel(x, I, atoms):
    raise NotImplementedError("write your pallas kernel here")


# speedup vs baseline: 14.2424x; 14.2424x over previous
import functools

import jax
import jax.numpy as jnp
from jax import lax
from jax.experimental import pallas as pl
from jax.experimental.pallas import tpu as pltpu
from jax.experimental.pallas import tpu_sc as plsc

GAMMA = 0.01
INV_GAMMA = 1.0 / GAMMA
INFER_STEP = 2

C, G, S, L = 4, 10000, 16, 4
B = 32
CG = C * G
SL = S * L

NC, NS = 2, 16
NW = NC * NS
ROWS_PER_W = CG // NW
CH = 25
NIT = ROWS_PER_W // CH
HALF = 16

_sc_mesh = plsc.VectorSubcoreMesh(
    core_axis_name="c", subcore_axis_name="s", num_cores=NC, num_subcores=NS)


def _sc_body(table_hbm, idx_hbm, m_hbm, u_hbm, idx_v, rows_v, m_buf, u_buf, sem):
    wid = lax.axis_index("s") * NC + lax.axis_index("c")
    w_base = wid * ROWS_PER_W

    def chunk_body(it, _):
        base = w_base + it * CH
        pltpu.sync_copy(idx_hbm.at[pl.ds(base * SL, CH * SL)], idx_v)
        pltpu.async_copy(table_hbm.at[idx_v], rows_v, sem).wait()

        def g_body(g, _):
            for h in range(2):
                lane = pl.ds(h * HALF, HALF)
                mns = []
                dns = []
                for s in range(S):
                    v = [rows_v[g * SL + s * L + l, lane] for l in range(L)]
                    mn = jnp.minimum(jnp.minimum(v[0], v[1]),
                                     jnp.minimum(v[2], v[3]))
                    dn = (jnp.exp((mn - v[0]) * INV_GAMMA)
                          + jnp.exp((mn - v[1]) * INV_GAMMA)
                          + jnp.exp((mn - v[2]) * INV_GAMMA)
                          + jnp.exp((mn - v[3]) * INV_GAMMA))
                    mns.append(mn)
                    dns.append(dn)
                t = list(mns)
                while len(t) > 1:
                    t = [jnp.maximum(t[i], t[i + 1]) for i in range(0, len(t) - 1, 2)] \
                        + ([t[-1]] if len(t) % 2 else [])
                m = t[0]
                u = jnp.exp((mns[0] - m) * INV_GAMMA) / dns[0]
                for s in range(1, S):
                    u = u + jnp.exp((mns[s] - m) * INV_GAMMA) / dns[s]
                m_buf[g, lane] = m
                u_buf[g, lane] = u
            return ()

        lax.fori_loop(0, CH, g_body, (), unroll=False)
        pltpu.sync_copy(m_buf, m_hbm.at[pl.ds(base, CH)])
        pltpu.sync_copy(u_buf, u_hbm.at[pl.ds(base, CH)])
        return ()

    lax.fori_loop(0, NIT, chunk_body, (), unroll=False)


@functools.partial(
    pl.kernel,
    out_type=(
        jax.ShapeDtypeStruct((CG, B), jnp.float32),
        jax.ShapeDtypeStruct((CG, B), jnp.float32),
    ),
    mesh=_sc_mesh,
    scratch_types=[
        pltpu.VMEM((CH * SL,), jnp.int32),
        pltpu.VMEM((CH * SL, B), jnp.float32),
        pltpu.VMEM((CH, B), jnp.float32),
        pltpu.VMEM((CH, B), jnp.float32),
        pltpu.SemaphoreType.DMA,
    ],
    compiler_params=pltpu.CompilerParams(use_tc_tiling_on_sc=False),
)
def _sc_clause_mu(table_hbm, idx_hbm, m_hbm, u_hbm, idx_v, rows_v, m_buf, u_buf, sem):
    _sc_body(table_hbm, idx_hbm, m_hbm, u_hbm, idx_v, rows_v, m_buf, u_buf, sem)


RW = CG * B // C // 128


def _tc_finalize_body(m_ref, u_ref, rp_ref, out_ref):
    lses = []
    mo = None
    for c in range(C):
        r = m_ref[c] + GAMMA * jnp.log(u_ref[c])
        mc = jnp.max(r)
        r = jnp.where(mc > 1.0, r / mc, r)
        rp = rp_ref[c]
        mx = jnp.maximum(rp, r)
        lse2 = mx + GAMMA * jnp.log(jnp.exp((rp - mx) * INV_GAMMA)
                                    + jnp.exp((r - mx) * INV_GAMMA))
        lses.append(lse2)
        mcur = jnp.max(lse2)
        mo = mcur if mo is None else jnp.maximum(mo, mcur)
    for c in range(C):
        out_ref[c] = jnp.where(mo > 1.0, lses[c] / mo, lses[c])


def _tc_finalize(m, u, rprev):
    return pl.pallas_call(
        _tc_finalize_body,
        out_shape=jax.ShapeDtypeStruct((C, RW, 128), jnp.float32),
    )(m.reshape(C, RW, 128), u.reshape(C, RW, 128), rprev.reshape(C, RW, 128))


def kernel(x, I, atoms):
    assert x.shape == (B, G) and I.shape == (C, G, S, L)
    xT = x.T
    offs = (jnp.arange(C, dtype=jnp.int32) * G)[:, None, None]
    idx = (I.astype(jnp.int32).reshape(C, G, SL) + offs).reshape(CG * SL)
    table = jnp.broadcast_to(xT[None], (C, G, B)).reshape(CG, B)
    rprev = table
    for _ in range(INFER_STEP):
        m, u = _sc_clause_mu(table, idx)
        rnew = _tc_finalize(m, u, rprev).reshape(CG, B)
        rprev = rnew
        table = rnew
    return jnp.swapaxes(rnew.reshape(C, G, B), 1, 2)


# speedup vs baseline: 16.8024x; 1.1797x over previous
import functools

import jax
import jax.numpy as jnp
from jax import lax
from jax.experimental import pallas as pl
from jax.experimental.pallas import tpu as pltpu
from jax.experimental.pallas import tpu_sc as plsc

GAMMA = 0.01
INV_GAMMA = 1.0 / GAMMA
INFER_STEP = 2

C, G, S, L = 4, 10000, 16, 4
B = 32
CG = C * G
SL = S * L

NC, NS = 2, 16
NW = NC * NS
ROWS_PER_W = CG // NW
CH = 25
NIT = ROWS_PER_W // CH
FLUSH = 5
HALF = 16

_sc_mesh = plsc.VectorSubcoreMesh(
    core_axis_name="c", subcore_axis_name="s", num_cores=NC, num_subcores=NS)


def _reduce_rows(rows_v, g, m_buf, u_buf, out_row):
    for h in range(2):
        lane = pl.ds(h * HALF, HALF)
        mns = []
        dns = []
        for s in range(S):
            v = [rows_v[g * SL + s * L + l, lane] for l in range(L)]
            mn = jnp.minimum(jnp.minimum(v[0], v[1]),
                             jnp.minimum(v[2], v[3]))
            dn = (jnp.exp((mn - v[0]) * INV_GAMMA)
                  + jnp.exp((mn - v[1]) * INV_GAMMA)
                  + jnp.exp((mn - v[2]) * INV_GAMMA)
                  + jnp.exp((mn - v[3]) * INV_GAMMA))
            mns.append(mn)
            dns.append(dn)
        t = list(mns)
        while len(t) > 1:
            t = [jnp.maximum(t[i], t[i + 1]) for i in range(0, len(t) - 1, 2)] \
                + ([t[-1]] if len(t) % 2 else [])
        m = t[0]
        u = jnp.exp((mns[0] - m) * INV_GAMMA) / dns[0]
        for s in range(1, S):
            u = u + jnp.exp((mns[s] - m) * INV_GAMMA) / dns[s]
        m_buf[out_row, lane] = m
        u_buf[out_row, lane] = u


def _sc_body(table_hbm, idx_hbm, m_hbm, u_hbm,
             idx0, idx1, rows0, rows1, m_buf, u_buf, sem0, sem1):
    wid = lax.axis_index("s") * NC + lax.axis_index("c")
    w_base = wid * ROWS_PER_W
    idx_v = (idx0, idx1)
    rows_v = (rows0, rows1)
    sems = (sem0, sem1)

    def stage_and_start(chunk, p):
        base = w_base + chunk * CH
        pltpu.sync_copy(idx_hbm.at[pl.ds(base * SL, CH * SL)], idx_v[p])
        pltpu.async_copy(table_hbm.at[idx_v[p]], rows_v[p], sems[p])

    stage_and_start(0, 0)

    def pair_body(pair, _):
        for p in range(2):
            chunk = pair * 2 + p

            @pl.when(chunk + 1 < NIT)
            def _():
                stage_and_start(chunk + 1, p ^ 1)

            pltpu.make_async_copy(
                table_hbm.at[idx_v[p]], rows_v[p], sems[p]).wait()
            off = (chunk % FLUSH) * CH

            def g_body(g, _):
                _reduce_rows(rows_v[p], g, m_buf, u_buf, off + g)
                return ()

            lax.fori_loop(0, CH, g_body, (), unroll=False)

            @pl.when(chunk % FLUSH == FLUSH - 1)
            def _():
                fb = w_base + (chunk - (FLUSH - 1)) * CH
                pltpu.sync_copy(m_buf, m_hbm.at[pl.ds(fb, FLUSH * CH)])
                pltpu.sync_copy(u_buf, u_hbm.at[pl.ds(fb, FLUSH * CH)])
        return ()

    lax.fori_loop(0, NIT // 2, pair_body, (), unroll=False)


def _make_sc_mu(table_rows):
    return functools.partial(
        pl.kernel,
        out_type=(
            jax.ShapeDtypeStruct((CG, B), jnp.float32),
            jax.ShapeDtypeStruct((CG, B), jnp.float32),
        ),
        mesh=_sc_mesh,
        scratch_types=[
            pltpu.VMEM((CH * SL,), jnp.int32),
            pltpu.VMEM((CH * SL,), jnp.int32),
            pltpu.VMEM((CH * SL, B), jnp.float32),
            pltpu.VMEM((CH * SL, B), jnp.float32),
            pltpu.VMEM((FLUSH * CH, B), jnp.float32),
            pltpu.VMEM((FLUSH * CH, B), jnp.float32),
            pltpu.SemaphoreType.DMA,
            pltpu.SemaphoreType.DMA,
        ],
        compiler_params=pltpu.CompilerParams(use_tc_tiling_on_sc=False),
    )(_sc_body)


_sc_mu_g = _make_sc_mu(G)
_sc_mu_cg = _make_sc_mu(CG)


RW = G * B // 128


def _tc_finalize_body(m_ref, u_ref, rp_ref, out_ref):
    lses = []
    mo = None
    for c in range(C):
        r = m_ref[c] + GAMMA * jnp.log(u_ref[c])
        mc = jnp.max(r)
        r = jnp.where(mc > 1.0, r / mc, r)
        rp = rp_ref[c] if rp_ref.shape[0] == C else rp_ref[0]
        mx = jnp.maximum(rp, r)
        lse2 = mx + GAMMA * jnp.log(jnp.exp((rp - mx) * INV_GAMMA)
                                    + jnp.exp((r - mx) * INV_GAMMA))
        lses.append(lse2)
        mcur = jnp.max(lse2)
        mo = mcur if mo is None else jnp.maximum(mo, mcur)
    for c in range(C):
        out_ref[c] = jnp.where(mo > 1.0, lses[c] / mo, lses[c])


def _tc_finalize(m, u, rprev):
    nrp = rprev.shape[0] * B // (RW * 128)
    return pl.pallas_call(
        _tc_finalize_body,
        out_shape=jax.ShapeDtypeStruct((C, RW, 128), jnp.float32),
    )(m.reshape(C, RW, 128), u.reshape(C, RW, 128),
      rprev.reshape(nrp, RW, 128))


def _tc_transpose_body(in_ref, out_ref):
    for c in range(C):
        out_ref[c] = jnp.transpose(in_ref[c], (1, 0))


def _tc_transpose(r):
    return pl.pallas_call(
        _tc_transpose_body,
        out_shape=jax.ShapeDtypeStruct((C, B, G), jnp.float32),
    )(r)


def kernel(x, I, atoms):
    assert x.shape == (B, G) and I.shape == (C, G, S, L)
    xT = x.T
    idx1 = I.astype(jnp.int32).reshape(CG * SL)
    offs = (jnp.arange(C, dtype=jnp.int32) * G)[:, None, None]
    idx2 = (I.astype(jnp.int32).reshape(C, G, SL) + offs).reshape(CG * SL)

    m, u = _sc_mu_g(xT, idx1)
    rnew = _tc_finalize(m, u, xT).reshape(CG, B)

    m, u = _sc_mu_cg(rnew, idx2)
    rnew = _tc_finalize(m, u, rnew)

    return _tc_transpose(rnew.reshape(C, G, B))


# speedup vs baseline: 21.5814x; 1.2844x over previous
import functools

import jax
import jax.numpy as jnp
from jax import lax
from jax.experimental import pallas as pl
from jax.experimental.pallas import tpu as pltpu
from jax.experimental.pallas import tpu_sc as plsc

GAMMA = 0.01
INV_GAMMA = 1.0 / GAMMA
INFER_STEP = 2

C, G, S, L = 4, 10000, 16, 4
B = 32
CG = C * G
SL = S * L
GP = 10240

NC, NS = 2, 16
NW = NC * NS
WPC = NW // C
GPW = GP // WPC
CH = 16
NIT = GPW // CH
FLUSH = 8
HALF = 16

_sc_mesh = plsc.VectorSubcoreMesh(
    core_axis_name="c", subcore_axis_name="s", num_cores=NC, num_subcores=NS)


def _reduce_rows(rows_v, g, m_buf, u_buf, out_row):
    for h in range(2):
        lane = pl.ds(h * HALF, HALF)
        mns = []
        dns = []
        for s in range(S):
            v = [rows_v[(s * L + l) * CH + g, lane] for l in range(L)]
            mn = jnp.minimum(jnp.minimum(v[0], v[1]),
                             jnp.minimum(v[2], v[3]))
            dn = (jnp.exp((mn - v[0]) * INV_GAMMA)
                  + jnp.exp((mn - v[1]) * INV_GAMMA)
                  + jnp.exp((mn - v[2]) * INV_GAMMA)
                  + jnp.exp((mn - v[3]) * INV_GAMMA))
            mns.append(mn)
            dns.append(dn)
        t = list(mns)
        while len(t) > 1:
            t = [jnp.maximum(t[i], t[i + 1]) for i in range(0, len(t) - 1, 2)] \
                + ([t[-1]] if len(t) % 2 else [])
        m = t[0]
        u = jnp.exp((mns[0] - m) * INV_GAMMA) / dns[0]
        for s in range(1, S):
            u = u + jnp.exp((mns[s] - m) * INV_GAMMA) / dns[s]
        m_buf[out_row, lane] = m
        u_buf[out_row, lane] = u


def _sc_body(table_hbm, idx_hbm, m_hbm, u_hbm,
             idx3d0, idx3d1, idxf0, idxf1, rows0, rows1,
             m_buf, u_buf, sem0, sem1):
    wid = lax.axis_index("s") * NC + lax.axis_index("c")
    c_w = wid // WPC
    g_base = (wid % WPC) * GPW
    row_off = c_w * G
    out_base = c_w * GP + g_base
    idx3d = (idx3d0, idx3d1)
    idxf = (idxf0, idxf1)
    rows_v = (rows0, rows1)
    sems = (sem0, sem1)

    def stage_and_start(chunk, p):
        g0 = g_base + chunk * CH
        pltpu.sync_copy(idx_hbm.at[c_w, :, :, pl.ds(g0, CH)], idx3d[p])
        for sl in range(S * L):
            s, l = sl // L, sl % L
            idxf[p][pl.ds(sl * CH, CH)] = idx3d[p][s, l, :] + row_off
        pltpu.async_copy(table_hbm.at[idxf[p]], rows_v[p], sems[p])

    stage_and_start(0, 0)

    def pair_body(pair, _):
        for p in range(2):
            chunk = pair * 2 + p

            @pl.when(chunk + 1 < NIT)
            def _():
                stage_and_start(chunk + 1, p ^ 1)

            pltpu.make_async_copy(
                table_hbm.at[idxf[p]], rows_v[p], sems[p]).wait()
            off = (chunk % FLUSH) * CH

            def g_body(g, _):
                _reduce_rows(rows_v[p], g, m_buf, u_buf, off + g)
                return ()

            lax.fori_loop(0, CH, g_body, (), unroll=False)

            @pl.when(chunk % FLUSH == FLUSH - 1)
            def _():
                fb = out_base + (chunk - (FLUSH - 1)) * CH
                pltpu.sync_copy(m_buf, m_hbm.at[pl.ds(fb, FLUSH * CH)])
                pltpu.sync_copy(u_buf, u_hbm.at[pl.ds(fb, FLUSH * CH)])
        return ()

    lax.fori_loop(0, NIT // 2, pair_body, (), unroll=False)


_sc_mu = functools.partial(
    pl.kernel,
    out_type=(
        jax.ShapeDtypeStruct((C * GP, B), jnp.float32),
        jax.ShapeDtypeStruct((C * GP, B), jnp.float32),
    ),
    mesh=_sc_mesh,
    scratch_types=[
        pltpu.VMEM((S, L, CH), jnp.int32),
        pltpu.VMEM((S, L, CH), jnp.int32),
        pltpu.VMEM((L * S * CH,), jnp.int32),
        pltpu.VMEM((L * S * CH,), jnp.int32),
        pltpu.VMEM((L * S * CH, B), jnp.float32),
        pltpu.VMEM((L * S * CH, B), jnp.float32),
        pltpu.VMEM((FLUSH * CH, B), jnp.float32),
        pltpu.VMEM((FLUSH * CH, B), jnp.float32),
        pltpu.SemaphoreType.DMA,
        pltpu.SemaphoreType.DMA,
    ],
    compiler_params=pltpu.CompilerParams(use_tc_tiling_on_sc=False),
)(_sc_body)


RW = G * B // 128
RWP = GP * B // 128


def _tc_finalize_body(m_ref, u_ref, rp_ref, out_ref):
    lses = []
    mo = None
    for c in range(C):
        r = m_ref[c, :RW] + GAMMA * jnp.log(u_ref[c, :RW])
        mc = jnp.max(r)
        r = jnp.where(mc > 1.0, r / mc, r)
        rp = rp_ref[c] if rp_ref.shape[0] == C else rp_ref[0]
        mx = jnp.maximum(rp, r)
        lse2 = mx + GAMMA * jnp.log(jnp.exp((rp - mx) * INV_GAMMA)
                                    + jnp.exp((r - mx) * INV_GAMMA))
        lses.append(lse2)
        mcur = jnp.max(lse2)
        mo = mcur if mo is None else jnp.maximum(mo, mcur)
    for c in range(C):
        out_ref[c] = jnp.where(mo > 1.0, lses[c] / mo, lses[c])


def _tc_finalize(m, u, rprev):
    nrp = rprev.shape[0] * B // (RW * 128)
    return pl.pallas_call(
        _tc_finalize_body,
        out_shape=jax.ShapeDtypeStruct((C, RW, 128), jnp.float32),
    )(m.reshape(C, RWP, 128), u.reshape(C, RWP, 128),
      rprev.reshape(nrp, RW, 128))


def _tc_transpose_body(in_ref, out_ref):
    for c in range(C):
        out_ref[c] = jnp.transpose(in_ref[c], (1, 0))


def _tc_transpose(r):
    return pl.pallas_call(
        _tc_transpose_body,
        out_shape=jax.ShapeDtypeStruct((C, B, G), jnp.float32),
    )(r)


def kernel(x, I, atoms):
    assert x.shape == (B, G) and I.shape == (C, G, S, L)
    xT = x.T
    idx = jnp.pad(jnp.transpose(I.astype(jnp.int32), (0, 2, 3, 1)),
                  ((0, 0), (0, 0), (0, 0), (0, GP - G)))
    table = jnp.broadcast_to(xT[None], (C, G, B)).reshape(CG, B)
    rprev = table

    m, u = _sc_mu(table, idx)
    rnew = _tc_finalize(m, u, rprev).reshape(CG, B)

    m, u = _sc_mu(rnew, idx)
    rnew = _tc_finalize(m, u, rnew)

    return _tc_transpose(rnew.reshape(C, G, B))


# speedup vs baseline: 23.0930x; 1.0700x over previous
import functools

import jax
import jax.numpy as jnp
from jax import lax
from jax.experimental import pallas as pl
from jax.experimental.pallas import tpu as pltpu
from jax.experimental.pallas import tpu_sc as plsc

GAMMA = 0.01
INV_GAMMA = 1.0 / GAMMA
INFER_STEP = 2

C, G, S, L = 4, 10000, 16, 4
B = 32
CG = C * G
SL = S * L
GP = 10240

NC, NS = 2, 16
NW = NC * NS
WPC = NW // C
GPW = GP // WPC
CH = 16
NIT = GPW // CH
FLUSH = 8
HALF = 16

_sc_mesh = plsc.VectorSubcoreMesh(
    core_axis_name="c", subcore_axis_name="s", num_cores=NC, num_subcores=NS)


def _reduce_rows(rows_v, g, m_buf, u_buf, out_row):
    for h in range(2):
        lane = pl.ds(h * HALF, HALF)
        mns = []
        dns = []
        for s in range(S):
            v = [rows_v[(s * L + l) * CH + g, lane] for l in range(L)]
            mn = jnp.minimum(jnp.minimum(v[0], v[1]),
                             jnp.minimum(v[2], v[3]))
            dn = (jnp.exp((mn - v[0]) * INV_GAMMA)
                  + jnp.exp((mn - v[1]) * INV_GAMMA)
                  + jnp.exp((mn - v[2]) * INV_GAMMA)
                  + jnp.exp((mn - v[3]) * INV_GAMMA))
            mns.append(mn)
            dns.append(dn)
        t = list(mns)
        while len(t) > 1:
            t = [jnp.maximum(t[i], t[i + 1]) for i in range(0, len(t) - 1, 2)] \
                + ([t[-1]] if len(t) % 2 else [])
        m = t[0]
        u = jnp.exp((mns[0] - m) * INV_GAMMA) / dns[0]
        for s in range(1, S):
            u = u + jnp.exp((mns[s] - m) * INV_GAMMA) / dns[s]
        m_buf[out_row, lane] = m
        u_buf[out_row, lane] = u


def _sc_body(table_hbm, idx_hbm, m_hbm, u_hbm,
             idx3d0, idx3d1, idxf0, idxf1, rows0, rows1,
             m_buf, u_buf, semr0, semr1, semi0, semi1):
    wid = lax.axis_index("s") * NC + lax.axis_index("c")
    c_w = wid // WPC
    g_base = (wid % WPC) * GPW
    row_off = c_w * G
    out_base = c_w * GP + g_base
    idx3d = (idx3d0, idx3d1)
    idxf = (idxf0, idxf1)
    rows_v = (rows0, rows1)
    semr = (semr0, semr1)
    semi = (semi0, semi1)

    def idx_copy(chunk, p):
        g0 = g_base + chunk * CH
        return pltpu.make_async_copy(
            idx_hbm.at[c_w, :, :, pl.ds(g0, CH)], idx3d[p], semi[p])

    def repack_and_gather(p):
        for sl in range(S * L):
            s, l = sl // L, sl % L
            idxf[p][pl.ds(sl * CH, CH)] = idx3d[p][s, l, :] + row_off
        pltpu.async_copy(table_hbm.at[idxf[p]], rows_v[p], semr[p])

    idx_copy(0, 0).start()
    idx_copy(0, 0).wait()
    repack_and_gather(0)
    idx_copy(1, 1).start()

    def pair_body(pair, _):
        for p in range(2):
            chunk = pair * 2 + p

            @pl.when(chunk + 1 < NIT)
            def _():
                idx_copy(chunk + 1, p ^ 1).wait()
                repack_and_gather(p ^ 1)

            @pl.when(chunk + 2 < NIT)
            def _():
                idx_copy(chunk + 2, p).start()

            pltpu.make_async_copy(
                table_hbm.at[idxf[p]], rows_v[p], semr[p]).wait()
            off = (chunk % FLUSH) * CH

            def g_body(g, _):
                _reduce_rows(rows_v[p], g, m_buf, u_buf, off + g)
                return ()

            lax.fori_loop(0, CH, g_body, (), unroll=False)

            @pl.when(chunk % FLUSH == FLUSH - 1)
            def _():
                fb = out_base + (chunk - (FLUSH - 1)) * CH
                pltpu.sync_copy(m_buf, m_hbm.at[pl.ds(fb, FLUSH * CH)])
                pltpu.sync_copy(u_buf, u_hbm.at[pl.ds(fb, FLUSH * CH)])
        return ()

    lax.fori_loop(0, NIT // 2, pair_body, (), unroll=False)


_sc_mu = functools.partial(
    pl.kernel,
    out_type=(
        jax.ShapeDtypeStruct((C * GP, B), jnp.float32),
        jax.ShapeDtypeStruct((C * GP, B), jnp.float32),
    ),
    mesh=_sc_mesh,
    scratch_types=[
        pltpu.VMEM((S, L, CH), jnp.int32),
        pltpu.VMEM((S, L, CH), jnp.int32),
        pltpu.VMEM((L * S * CH,), jnp.int32),
        pltpu.VMEM((L * S * CH,), jnp.int32),
        pltpu.VMEM((L * S * CH, B), jnp.float32),
        pltpu.VMEM((L * S * CH, B), jnp.float32),
        pltpu.VMEM((FLUSH * CH, B), jnp.float32),
        pltpu.VMEM((FLUSH * CH, B), jnp.float32),
        pltpu.SemaphoreType.DMA,
        pltpu.SemaphoreType.DMA,
        pltpu.SemaphoreType.DMA,
        pltpu.SemaphoreType.DMA,
    ],
    compiler_params=pltpu.CompilerParams(use_tc_tiling_on_sc=False),
)(_sc_body)


RW = G * B // 128
RWP = GP * B // 128


def _tc_finalize_body(m_ref, u_ref, rp_ref, out_ref):
    lses = []
    mo = None
    for c in range(C):
        r = m_ref[c, :RW] + GAMMA * jnp.log(u_ref[c, :RW])
        mc = jnp.max(r)
        r = jnp.where(mc > 1.0, r / mc, r)
        rp = rp_ref[c] if rp_ref.shape[0] == C else rp_ref[0]
        mx = jnp.maximum(rp, r)
        lse2 = mx + GAMMA * jnp.log(jnp.exp((rp - mx) * INV_GAMMA)
                                    + jnp.exp((r - mx) * INV_GAMMA))
        lses.append(lse2)
        mcur = jnp.max(lse2)
        mo = mcur if mo is None else jnp.maximum(mo, mcur)
    for c in range(C):
        out_ref[c] = jnp.where(mo > 1.0, lses[c] / mo, lses[c])


def _tc_finalize(m, u, rprev):
    nrp = rprev.shape[0] * B // (RW * 128)
    return pl.pallas_call(
        _tc_finalize_body,
        out_shape=jax.ShapeDtypeStruct((C, RW, 128), jnp.float32),
    )(m.reshape(C, RWP, 128), u.reshape(C, RWP, 128),
      rprev.reshape(nrp, RW, 128))


def _tc_transpose_body(in_ref, out_ref):
    for c in range(C):
        out_ref[c] = jnp.transpose(in_ref[c], (1, 0))


def _tc_transpose(r):
    return pl.pallas_call(
        _tc_transpose_body,
        out_shape=jax.ShapeDtypeStruct((C, B, G), jnp.float32),
    )(r)


def kernel(x, I, atoms):
    assert x.shape == (B, G) and I.shape == (C, G, S, L)
    xT = x.T
    idx = jnp.pad(jnp.transpose(I.astype(jnp.int32), (0, 2, 3, 1)),
                  ((0, 0), (0, 0), (0, 0), (0, GP - G)))
    table = jnp.broadcast_to(xT[None], (C, G, B)).reshape(CG, B)
    rprev = table

    m, u = _sc_mu(table, idx)
    rnew = _tc_finalize(m, u, rprev).reshape(CG, B)

    m, u = _sc_mu(rnew, idx)
    rnew = _tc_finalize(m, u, rnew)

    return _tc_transpose(rnew.reshape(C, G, B))


# speedup vs baseline: 24.1568x; 1.0461x over previous
import functools

import jax
import jax.numpy as jnp
from jax import lax
from jax.experimental import pallas as pl
from jax.experimental.pallas import tpu as pltpu
from jax.experimental.pallas import tpu_sc as plsc

GAMMA = 0.01
INV_GAMMA = 1.0 / GAMMA
INFER_STEP = 2

C, G, S, L = 4, 10000, 16, 4
B = 32
CG = C * G
SL = S * L
GP = 10240

NC, NS = 2, 16
NW = NC * NS
WPC = NW // C
GPW = GP // WPC
CH = 16
NIT = GPW // CH
FLUSH = 8
HALF = 16

_sc_mesh = plsc.VectorSubcoreMesh(
    core_axis_name="c", subcore_axis_name="s", num_cores=NC, num_subcores=NS)


def _reduce_rows(rows_v, g, m_buf, u_buf, out_row):
    for h in range(2):
        lane = pl.ds(h * HALF, HALF)
        mns = []
        dns = []
        for s in range(S):
            v = [rows_v[(s * L + l) * CH + g, lane] for l in range(L)]
            mn = jnp.minimum(jnp.minimum(v[0], v[1]),
                             jnp.minimum(v[2], v[3]))
            dn = (jnp.exp((mn - v[0]) * INV_GAMMA)
                  + jnp.exp((mn - v[1]) * INV_GAMMA)
                  + jnp.exp((mn - v[2]) * INV_GAMMA)
                  + jnp.exp((mn - v[3]) * INV_GAMMA))
            mns.append(mn)
            dns.append(dn)
        t = list(mns)
        while len(t) > 1:
            t = [jnp.maximum(t[i], t[i + 1]) for i in range(0, len(t) - 1, 2)] \
                + ([t[-1]] if len(t) % 2 else [])
        m = t[0]
        u = jnp.exp((mns[0] - m) * INV_GAMMA) / dns[0]
        for s in range(1, S):
            u = u + jnp.exp((mns[s] - m) * INV_GAMMA) / dns[s]
        m_buf[out_row, lane] = m
        u_buf[out_row, lane] = u


def _sc_body(table_hbm, idx_hbm, m_hbm, u_hbm,
             idx3d0, idx3d1, idxf0, idxf1, rows0, rows1,
             m_buf, u_buf, semr0, semr1, semi0, semi1):
    sid = lax.axis_index("s")
    cid = lax.axis_index("c")
    wid = sid * NC + (sid + cid) % NC
    c_w = wid // WPC
    g_base = (wid % WPC) * GPW
    row_off = c_w * G
    out_base = c_w * GP + g_base
    idx3d = (idx3d0, idx3d1)
    idxf = (idxf0, idxf1)
    rows_v = (rows0, rows1)
    semr = (semr0, semr1)
    semi = (semi0, semi1)

    def idx_copy(chunk, p):
        g0 = g_base + chunk * CH
        return pltpu.make_async_copy(
            idx_hbm.at[c_w, :, :, pl.ds(g0, CH)], idx3d[p], semi[p])

    def repack_and_gather(p):
        for sl in range(S * L):
            s, l = sl // L, sl % L
            idxf[p][pl.ds(sl * CH, CH)] = idx3d[p][s, l, :] + row_off
        pltpu.async_copy(table_hbm.at[idxf[p]], rows_v[p], semr[p])

    idx_copy(0, 0).start()
    idx_copy(0, 0).wait()
    repack_and_gather(0)
    idx_copy(1, 1).start()

    def pair_body(pair, _):
        for p in range(2):
            chunk = pair * 2 + p

            @pl.when(chunk + 1 < NIT)
            def _():
                idx_copy(chunk + 1, p ^ 1).wait()
                repack_and_gather(p ^ 1)

            @pl.when(chunk + 2 < NIT)
            def _():
                idx_copy(chunk + 2, p).start()

            pltpu.make_async_copy(
                table_hbm.at[idxf[p]], rows_v[p], semr[p]).wait()
            off = (chunk % FLUSH) * CH

            def g_body(g, _):
                _reduce_rows(rows_v[p], g, m_buf, u_buf, off + g)
                return ()

            lax.fori_loop(0, CH, g_body, (), unroll=False)

            @pl.when(chunk % FLUSH == FLUSH - 1)
            def _():
                fb = out_base + (chunk - (FLUSH - 1)) * CH
                pltpu.sync_copy(m_buf, m_hbm.at[pl.ds(fb, FLUSH * CH)])
                pltpu.sync_copy(u_buf, u_hbm.at[pl.ds(fb, FLUSH * CH)])
        return ()

    lax.fori_loop(0, NIT // 2, pair_body, (), unroll=False)


_sc_mu = functools.partial(
    pl.kernel,
    out_type=(
        jax.ShapeDtypeStruct((C * GP, B), jnp.float32),
        jax.ShapeDtypeStruct((C * GP, B), jnp.float32),
    ),
    mesh=_sc_mesh,
    scratch_types=[
        pltpu.VMEM((S, L, CH), jnp.int32),
        pltpu.VMEM((S, L, CH), jnp.int32),
        pltpu.VMEM((L * S * CH,), jnp.int32),
        pltpu.VMEM((L * S * CH,), jnp.int32),
        pltpu.VMEM((L * S * CH, B), jnp.float32),
        pltpu.VMEM((L * S * CH, B), jnp.float32),
        pltpu.VMEM((FLUSH * CH, B), jnp.float32),
        pltpu.VMEM((FLUSH * CH, B), jnp.float32),
        pltpu.SemaphoreType.DMA,
        pltpu.SemaphoreType.DMA,
        pltpu.SemaphoreType.DMA,
        pltpu.SemaphoreType.DMA,
    ],
    compiler_params=pltpu.CompilerParams(use_tc_tiling_on_sc=False),
)(_sc_body)


RW = G * B // 128
RWP = GP * B // 128


def _tc_finalize_body(m_ref, u_ref, rp_ref, out_ref):
    lses = []
    mo = None
    for c in range(C):
        r = m_ref[c, :RW] + GAMMA * jnp.log(u_ref[c, :RW])
        mc = jnp.max(r)
        r = jnp.where(mc > 1.0, r / mc, r)
        rp = rp_ref[c] if rp_ref.shape[0] == C else rp_ref[0]
        mx = jnp.maximum(rp, r)
        lse2 = mx + GAMMA * jnp.log(jnp.exp((rp - mx) * INV_GAMMA)
                                    + jnp.exp((r - mx) * INV_GAMMA))
        lses.append(lse2)
        mcur = jnp.max(lse2)
        mo = mcur if mo is None else jnp.maximum(mo, mcur)
    for c in range(C):
        out_ref[c] = jnp.where(mo > 1.0, lses[c] / mo, lses[c])


def _tc_finalize(m, u, rprev):
    nrp = rprev.shape[0] * B // (RW * 128)
    return pl.pallas_call(
        _tc_finalize_body,
        out_shape=jax.ShapeDtypeStruct((C, RW, 128), jnp.float32),
    )(m.reshape(C, RWP, 128), u.reshape(C, RWP, 128),
      rprev.reshape(nrp, RW, 128))


def _tc_transpose_body(in_ref, out_ref):
    for c in range(C):
        out_ref[c] = jnp.transpose(in_ref[c], (1, 0))


def _tc_transpose(r):
    return pl.pallas_call(
        _tc_transpose_body,
        out_shape=jax.ShapeDtypeStruct((C, B, G), jnp.float32),
    )(r)


def kernel(x, I, atoms):
    assert x.shape == (B, G) and I.shape == (C, G, S, L)
    xT = x.T
    idx = jnp.pad(jnp.transpose(I.astype(jnp.int32), (0, 2, 3, 1)),
                  ((0, 0), (0, 0), (0, 0), (0, GP - G)))
    table = jnp.broadcast_to(xT[None], (C, G, B)).reshape(CG, B)
    rprev = table

    m, u = _sc_mu(table, idx)
    rnew = _tc_finalize(m, u, rprev).reshape(CG, B)

    m, u = _sc_mu(rnew, idx)
    rnew = _tc_finalize(m, u, rnew)

    return _tc_transpose(rnew.reshape(C, G, B))
